# baseline (speedup 1.0000x reference)
import jax
import jax.numpy as jnp
from jax.experimental import pallas as pl
from jax.experimental.pallas import tpu as pltpu

_REINIT_P = 0.05
_N, _H, _W = 10000, 250, 100
_B = 128
_S, _L = 8, 3125
_NCHUNK = 16


def _copy_body(buf_hbm, ns_hbm, newbuf_hbm, newns_hbm, sems):
    rows = _N // _NCHUNK
    copies = []
    for c in range(_NCHUNK):
        sl = pl.ds(c * rows, rows)
        copies.append(
            pltpu.make_async_copy(buf_hbm.at[sl], newbuf_hbm.at[sl], sems.at[c])
        )
    copies.append(pltpu.make_async_copy(ns_hbm, newns_hbm, sems.at[_NCHUNK]))
    for cp in copies:
        cp.start()
    for cp in copies:
        cp.wait()


def _gs_body(idx_ref, u_ref, buf_row, noise_row, ns_cell, newbuf_hbm, newns_hbm,
             out_row, newbuf_row, outns_cell, newns_cell):
    del newbuf_hbm, newns_hbm
    b = pl.program_id(0)
    reinit = u_ref[b] < _REINIT_P

    @pl.when(reinit)
    def _():
        out_row[...] = noise_row[...]
        newbuf_row[...] = noise_row[...]
        zero = jnp.zeros((1, 1, 1), jnp.float32)
        outns_cell[...] = zero
        newns_cell[...] = zero

    @pl.when(jnp.logical_not(reinit))
    def _():
        out_row[...] = buf_row[...]
        newbuf_row[...] = buf_row[...]
        outns_cell[...] = ns_cell[...]
        newns_cell[...] = ns_cell[...]


def kernel(buffer, buffer_numsteps, noise, u, idx):
    idx = idx.astype(jnp.int32)
    bufr = buffer.reshape(_N, _S, _L)
    noiser = noise.reshape(_B, _S, _L)
    ns3 = buffer_numsteps.reshape(_N, 1, 1)

    newbuf0, newns0 = pl.pallas_call(
        _copy_body,
        out_shape=[
            jax.ShapeDtypeStruct((_N, _S, _L), jnp.float32),
            jax.ShapeDtypeStruct((_N, 1, 1), jnp.float32),
        ],
        in_specs=[
            pl.BlockSpec(memory_space=pl.ANY),
            pl.BlockSpec(memory_space=pl.ANY),
        ],
        out_specs=[
            pl.BlockSpec(memory_space=pl.ANY),
            pl.BlockSpec(memory_space=pl.ANY),
        ],
        scratch_shapes=[pltpu.SemaphoreType.DMA((_NCHUNK + 1,))],
    )(bufr, ns3)

    grid_spec = pltpu.PrefetchScalarGridSpec(
        num_scalar_prefetch=2,
        grid=(_B,),
        in_specs=[
            pl.BlockSpec((1, _S, _L), lambda b, idx_r, u_r: (idx_r[b], 0, 0)),
            pl.BlockSpec((1, _S, _L), lambda b, idx_r, u_r: (b, 0, 0)),
            pl.BlockSpec((1, 1, 1), lambda b, idx_r, u_r: (idx_r[b], 0, 0)),
            pl.BlockSpec(memory_space=pl.ANY),
            pl.BlockSpec(memory_space=pl.ANY),
        ],
        out_specs=[
            pl.BlockSpec((1, _S, _L), lambda b, idx_r, u_r: (b, 0, 0)),
            pl.BlockSpec((1, _S, _L), lambda b, idx_r, u_r: (idx_r[b], 0, 0)),
            pl.BlockSpec((1, 1, 1), lambda b, idx_r, u_r: (b, 0, 0)),
            pl.BlockSpec((1, 1, 1), lambda b, idx_r, u_r: (idx_r[b], 0, 0)),
        ],
    )
    out_r, newbuf, outns, newns = pl.pallas_call(
        _gs_body,
        grid_spec=grid_spec,
        out_shape=[
            jax.ShapeDtypeStruct((_B, _S, _L), jnp.float32),
            jax.ShapeDtypeStruct((_N, _S, _L), jnp.float32),
            jax.ShapeDtypeStruct((_B, 1, 1), jnp.float32),
            jax.ShapeDtypeStruct((_N, 1, 1), jnp.float32),
        ],
        input_output_aliases={5: 1, 6: 3},
    )(idx, u, bufr, noiser, ns3, newbuf0, newns0)

    return (
        out_r.reshape(_B, _H, _W),
        outns.reshape(_B),
        newbuf.reshape(_N, _H, _W),
        newns.reshape(_N),
    )


# speedup vs baseline: 11.3395x; 11.3395x over previous
import jax
import jax.numpy as jnp
from jax.experimental import pallas as pl
from jax.experimental.pallas import tpu as pltpu

_REINIT_P = 0.05
_N, _H, _W = 10000, 250, 100
_B = 128
_R = 40


def _copy_body(buf_blk, ns_blk, newbuf_blk, newns_blk):
    newbuf_blk[...] = buf_blk[...]
    newns_blk[...] = ns_blk[...]


def _gs_body(idx_ref, u_ref, buf_row, noise_row, ns_cell, newbuf_hbm, newns_hbm,
             out_row, newbuf_row, outns_cell, newns_cell):
    del newbuf_hbm, newns_hbm
    b = pl.program_id(0)
    reinit = u_ref[b] < _REINIT_P

    @pl.when(reinit)
    def _():
        out_row[...] = noise_row[...]
        newbuf_row[...] = noise_row[...]
        zero = jnp.zeros((1, 1, 1), jnp.float32)
        outns_cell[...] = zero
        newns_cell[...] = zero

    @pl.when(jnp.logical_not(reinit))
    def _():
        out_row[...] = buf_row[...]
        newbuf_row[...] = buf_row[...]
        outns_cell[...] = ns_cell[...]
        newns_cell[...] = ns_cell[...]


def kernel(buffer, buffer_numsteps, noise, u, idx):
    idx = idx.astype(jnp.int32)
    ns3 = buffer_numsteps.reshape(_N, 1, 1)

    nsc = _R
    newbuf0, newns0 = pl.pallas_call(
        _copy_body,
        grid=(_N // _R,),
        out_shape=[
            jax.ShapeDtypeStruct((_N, _H, _W), jnp.float32),
            jax.ShapeDtypeStruct((_N, 1, 1), jnp.float32),
        ],
        in_specs=[
            pl.BlockSpec((_R, _H, _W), lambda i: (i, 0, 0)),
            pl.BlockSpec((nsc, 1, 1), lambda i: (i, 0, 0)),
        ],
        out_specs=[
            pl.BlockSpec((_R, _H, _W), lambda i: (i, 0, 0)),
            pl.BlockSpec((nsc, 1, 1), lambda i: (i, 0, 0)),
        ],
    )(buffer, ns3)

    grid_spec = pltpu.PrefetchScalarGridSpec(
        num_scalar_prefetch=2,
        grid=(_B,),
        in_specs=[
            pl.BlockSpec((1, _H, _W), lambda b, idx_r, u_r: (idx_r[b], 0, 0)),
            pl.BlockSpec((1, _H, _W), lambda b, idx_r, u_r: (b, 0, 0)),
            pl.BlockSpec((1, 1, 1), lambda b, idx_r, u_r: (idx_r[b], 0, 0)),
            pl.BlockSpec(memory_space=pl.ANY),
            pl.BlockSpec(memory_space=pl.ANY),
        ],
        out_specs=[
            pl.BlockSpec((1, _H, _W), lambda b, idx_r, u_r: (b, 0, 0)),
            pl.BlockSpec((1, _H, _W), lambda b, idx_r, u_r: (idx_r[b], 0, 0)),
            pl.BlockSpec((1, 1, 1), lambda b, idx_r, u_r: (b, 0, 0)),
            pl.BlockSpec((1, 1, 1), lambda b, idx_r, u_r: (idx_r[b], 0, 0)),
        ],
    )
    out, newbuf, outns, newns = pl.pallas_call(
        _gs_body,
        grid_spec=grid_spec,
        out_shape=[
            jax.ShapeDtypeStruct((_B, _H, _W), jnp.float32),
            jax.ShapeDtypeStruct((_N, _H, _W), jnp.float32),
            jax.ShapeDtypeStruct((_B, 1, 1), jnp.float32),
            jax.ShapeDtypeStruct((_N, 1, 1), jnp.float32),
        ],
        input_output_aliases={5: 1, 6: 3},
    )(idx, u, buffer, noise, ns3, newbuf0, newns0)

    return (out, outns.reshape(_B), newbuf, newns.reshape(_N))


# speedup vs baseline: 11.6366x; 1.0262x over previous
import jax
import jax.numpy as jnp
from jax.experimental import pallas as pl
from jax.experimental.pallas import tpu as pltpu

_REINIT_P = 0.05
_N, _H, _W = 10000, 250, 100
_B = 128
_R = 80
_NR, _NC = 80, 125


def _copy_body(buf_blk, newbuf_blk):
    newbuf_blk[...] = buf_blk[...]


def _gs_body(idx_ref, u_ref, buf_row, noise_row, newbuf_hbm,
             out_row, newbuf_row):
    del newbuf_hbm
    b = pl.program_id(0)
    reinit = u_ref[b] < _REINIT_P

    @pl.when(reinit)
    def _():
        out_row[...] = noise_row[...]
        newbuf_row[...] = noise_row[...]

    @pl.when(jnp.logical_not(reinit))
    def _():
        out_row[...] = buf_row[...]
        newbuf_row[...] = buf_row[...]


def _ns_body(idx_ref, u_ref, ns_ref, outns_ref, newns_ref):
    ns = ns_ref[...]
    rowid = jax.lax.broadcasted_iota(jnp.int32, (_NR, _NC), 0) * _NC + \
        jax.lax.broadcasted_iota(jnp.int32, (_NR, _NC), 1)
    bid = jax.lax.broadcasted_iota(jnp.int32, (1, _B), 1)

    def body(b, carry):
        acc, newns = carry
        i = idx_ref[b]
        reinit = u_ref[b] < _REINIT_P
        m = rowid == i
        val = jnp.where(reinit, 0.0, jnp.sum(jnp.where(m, ns, 0.0)))
        acc = jnp.where(bid == b, val, acc)
        newns = jnp.where(m, val, newns)
        return acc, newns

    acc, newns = jax.lax.fori_loop(
        0, _B, body, (jnp.zeros((1, _B), jnp.float32), ns))
    outns_ref[...] = acc
    newns_ref[...] = newns


def kernel(buffer, buffer_numsteps, noise, u, idx):
    idx = idx.astype(jnp.int32)

    newbuf0 = pl.pallas_call(
        _copy_body,
        grid=(_N // _R,),
        out_shape=jax.ShapeDtypeStruct((_N, _H, _W), jnp.float32),
        in_specs=[pl.BlockSpec((_R, _H, _W), lambda i: (i, 0, 0))],
        out_specs=pl.BlockSpec((_R, _H, _W), lambda i: (i, 0, 0)),
    )(buffer)

    grid_spec = pltpu.PrefetchScalarGridSpec(
        num_scalar_prefetch=2,
        grid=(_B,),
        in_specs=[
            pl.BlockSpec((1, _H, _W), lambda b, idx_r, u_r: (idx_r[b], 0, 0)),
            pl.BlockSpec((1, _H, _W), lambda b, idx_r, u_r: (b, 0, 0)),
            pl.BlockSpec(memory_space=pl.ANY),
        ],
        out_specs=[
            pl.BlockSpec((1, _H, _W), lambda b, idx_r, u_r: (b, 0, 0)),
            pl.BlockSpec((1, _H, _W), lambda b, idx_r, u_r: (idx_r[b], 0, 0)),
        ],
    )
    out, newbuf = pl.pallas_call(
        _gs_body,
        grid_spec=grid_spec,
        out_shape=[
            jax.ShapeDtypeStruct((_B, _H, _W), jnp.float32),
            jax.ShapeDtypeStruct((_N, _H, _W), jnp.float32),
        ],
        input_output_aliases={4: 1},
    )(idx, u, buffer, noise, newbuf0)

    outns, newns = pl.pallas_call(
        _ns_body,
        grid_spec=pltpu.PrefetchScalarGridSpec(
            num_scalar_prefetch=2,
            grid=(1,),
            in_specs=[pl.BlockSpec((_NR, _NC), lambda i, idx_r, u_r: (0, 0))],
            out_specs=[
                pl.BlockSpec((1, _B), lambda i, idx_r, u_r: (0, 0)),
                pl.BlockSpec((_NR, _NC), lambda i, idx_r, u_r: (0, 0)),
            ],
        ),
        out_shape=[
            jax.ShapeDtypeStruct((1, _B), jnp.float32),
            jax.ShapeDtypeStruct((_NR, _NC), jnp.float32),
        ],
    )(idx, u, buffer_numsteps.reshape(_NR, _NC))

    return (out, outns.reshape(_B), newbuf, newns.reshape(_N))


# speedup vs baseline: 11.6469x; 1.0009x over previous
import jax
import jax.numpy as jnp
from jax.experimental import pallas as pl
from jax.experimental.pallas import tpu as pltpu

_REINIT_P = 0.05
_N, _H, _W = 10000, 250, 100
_B = 128
_R = 80
_NR, _NC = 80, 125


_K = 8
_RC = 25
_NG = _N // (_K * _RC)


def _copy_body(buf_hbm, newbuf_hbm, slots, in_sems, out_sems):
    def in_copy(c, k):
        sl = pl.ds(c * _RC, _RC)
        return pltpu.make_async_copy(buf_hbm.at[sl], slots.at[k], in_sems.at[k])

    def out_copy(c, k):
        sl = pl.ds(c * _RC, _RC)
        return pltpu.make_async_copy(slots.at[k], newbuf_hbm.at[sl], out_sems.at[k])

    def group(g, _):
        for k in range(_K):
            c = g * _K + k

            @pl.when(g > 0)
            def _():
                out_copy(c - _K, k).wait()

            in_copy(c, k).start()
        for k in range(_K):
            c = g * _K + k
            in_copy(c, k).wait()
            out_copy(c, k).start()
        return 0

    jax.lax.fori_loop(0, _NG, group, 0)
    for k in range(_K):
        c = (_NG - 1) * _K + k
        out_copy(c, k).wait()


def _gs_body(idx_ref, u_ref, buf_row, noise_row, newbuf_hbm,
             out_row, newbuf_row):
    del newbuf_hbm
    b = pl.program_id(0)
    reinit = u_ref[b] < _REINIT_P

    @pl.when(reinit)
    def _():
        out_row[...] = noise_row[...]
        newbuf_row[...] = noise_row[...]

    @pl.when(jnp.logical_not(reinit))
    def _():
        out_row[...] = buf_row[...]
        newbuf_row[...] = buf_row[...]


def _ns_body(idx_ref, u_ref, ns_ref, outns_ref, newns_ref):
    ns = ns_ref[...]
    rowid = jax.lax.broadcasted_iota(jnp.int32, (_NR, _NC), 0) * _NC + \
        jax.lax.broadcasted_iota(jnp.int32, (_NR, _NC), 1)
    bid = jax.lax.broadcasted_iota(jnp.int32, (1, _B), 1)

    def body(b, carry):
        acc, newns = carry
        i = idx_ref[b]
        reinit = u_ref[b] < _REINIT_P
        m = rowid == i
        val = jnp.where(reinit, 0.0, jnp.sum(jnp.where(m, ns, 0.0)))
        acc = jnp.where(bid == b, val, acc)
        newns = jnp.where(m, val, newns)
        return acc, newns

    acc, newns = jax.lax.fori_loop(
        0, _B, body, (jnp.zeros((1, _B), jnp.float32), ns))
    outns_ref[...] = acc
    newns_ref[...] = newns


def kernel(buffer, buffer_numsteps, noise, u, idx):
    idx = idx.astype(jnp.int32)

    newbuf0 = pl.pallas_call(
        _copy_body,
        out_shape=jax.ShapeDtypeStruct((_N, _H, _W), jnp.float32),
        in_specs=[pl.BlockSpec(memory_space=pl.ANY)],
        out_specs=pl.BlockSpec(memory_space=pl.ANY),
        scratch_shapes=[
            pltpu.VMEM((_K, _RC, _H, _W), jnp.float32),
            pltpu.SemaphoreType.DMA((_K,)),
            pltpu.SemaphoreType.DMA((_K,)),
        ],
    )(buffer)

    grid_spec = pltpu.PrefetchScalarGridSpec(
        num_scalar_prefetch=2,
        grid=(_B,),
        in_specs=[
            pl.BlockSpec((1, _H, _W), lambda b, idx_r, u_r: (idx_r[b], 0, 0)),
            pl.BlockSpec((1, _H, _W), lambda b, idx_r, u_r: (b, 0, 0)),
            pl.BlockSpec(memory_space=pl.ANY),
        ],
        out_specs=[
            pl.BlockSpec((1, _H, _W), lambda b, idx_r, u_r: (b, 0, 0)),
            pl.BlockSpec((1, _H, _W), lambda b, idx_r, u_r: (idx_r[b], 0, 0)),
        ],
    )
    out, newbuf = pl.pallas_call(
        _gs_body,
        grid_spec=grid_spec,
        out_shape=[
            jax.ShapeDtypeStruct((_B, _H, _W), jnp.float32),
            jax.ShapeDtypeStruct((_N, _H, _W), jnp.float32),
        ],
        input_output_aliases={4: 1},
    )(idx, u, buffer, noise, newbuf0)

    outns, newns = pl.pallas_call(
        _ns_body,
        grid_spec=pltpu.PrefetchScalarGridSpec(
            num_scalar_prefetch=2,
            grid=(1,),
            in_specs=[pl.BlockSpec((_NR, _NC), lambda i, idx_r, u_r: (0, 0))],
            out_specs=[
                pl.BlockSpec((1, _B), lambda i, idx_r, u_r: (0, 0)),
                pl.BlockSpec((_NR, _NC), lambda i, idx_r, u_r: (0, 0)),
            ],
        ),
        out_shape=[
            jax.ShapeDtypeStruct((1, _B), jnp.float32),
            jax.ShapeDtypeStruct((_NR, _NC), jnp.float32),
        ],
    )(idx, u, buffer_numsteps.reshape(_NR, _NC))

    return (out, outns.reshape(_B), newbuf, newns.reshape(_N))
